# baseline (speedup 1.0000x reference)
import jax
import jax.numpy as jnp
from jax import lax
from jax.experimental import pallas as pl
from jax.experimental.pallas import tpu as pltpu
from jax.experimental.pallas import tpu_sc as plsc

_K = 64
_L = 16
_NC = 2
_NS = 16
_NW = _NC * _NS
_SLOT = 32
_CAP = _SLOT * _L
_CCAP = 256
_NSEG = 8


def _sc_body(x_hbm, out_hbm, row_v, obuf_v, grp_v, cidx_v, cidx2_v, cval2_v,
             pidx_v, sem_in, sem_out):
    n_rows, n_cols = x_hbm.shape
    n_vregs = n_cols // _L
    rows_per_w = n_rows // _NW
    seg_vregs = n_vregs // _NSEG
    ncand = _CCAP // _L

    wid = lax.axis_index("s") * _NC + lax.axis_index("c")
    row0 = wid * rows_per_w
    iota = lax.iota(jnp.int32, _L)
    zero_f = jnp.zeros((_L,), jnp.float32)
    zero_i = jnp.zeros((_L,), jnp.int32)

    pltpu.make_async_copy(
        x_hbm.at[row0], row_v.at[pl.ds(0, n_cols)], sem_in
    ).start()

    @plsc.parallel_loop(0, n_vregs, unroll=8)
    def _(i):
        obuf_v[pl.ds(i * _L, _L)] = zero_f

    def do_row(r, t_prev):
        row = row0 + r
        base = (r % 2) * n_cols
        nbase = ((r + 1) % 2) * n_cols
        base_splat = zero_i + base
        pltpu.make_async_copy(
            x_hbm.at[row], row_v.at[pl.ds(base, n_cols)], sem_in
        ).wait()

        @pl.when(r < rows_per_w - 1)
        def _():
            pltpu.make_async_copy(
                x_hbm.at[row + 1], row_v.at[pl.ds(nbase, n_cols)], sem_in
            ).start()

        def collect(mthr):
            for c in range(_CAP // _L):
                cidx_v[pl.ds(c * _L, _L)] = zero_i

            @plsc.parallel_loop(0, n_vregs, unroll=8, carry=zero_i)
            def cl(j, cl):
                v = row_v[pl.ds(base + j * _L, _L)]
                m = v >= mthr
                pos = jnp.minimum(cl, _SLOT - 1) * _L + iota
                plsc.store_scatter(cidx_v, [pos], iota + j * _L, mask=m)
                return cl + m.astype(jnp.int32)

            for c in range(ncand):
                cidx2_v[pl.ds(c * _L, _L)] = zero_i
                cval2_v[pl.ds(c * _L, _L)] = zero_f

            @plsc.parallel_loop(0, _SLOT, unroll=4, carry=zero_i)
            def cnt2(s, cnt):
                iv = cidx_v[pl.ds(s * _L, _L)]
                valid = cl > s
                vals = plsc.load_gather(row_v, [iv + base_splat])
                pos = cnt + plsc.cumsum(valid.astype(jnp.int32)) - 1
                pos = jnp.minimum(jnp.maximum(pos, 0), _CCAP - 1)
                plsc.store_scatter(cidx2_v, [pos], iv, mask=valid)
                plsc.store_scatter(cval2_v, [pos], vals, mask=valid)
                return cnt + plsc.all_reduce_population_count(valid)

            return cl, cnt2

        m_est = lax.bitcast_convert_type(t_prev, jnp.float32) * 0.9375
        cl, cnt2 = collect(m_est)

        max_cl = jnp.max(cl)
        n_cand = jnp.max(cnt2)
        fb = (max_cl >= _SLOT) | (n_cand < _K) | (n_cand > _CCAP)

        @pl.when(fb)
        def _():
            for seg in range(_NSEG):
                @plsc.parallel_loop(0, seg_vregs, unroll=8, carry=zero_f)
                def acc(i, a):
                    v = row_v[pl.ds(base + (seg * seg_vregs + i) * _L, _L)]
                    return jnp.maximum(a, v)
                grp_v[pl.ds(seg * _L, _L)] = acc

            gi = [
                lax.bitcast_convert_type(grp_v[pl.ds(k * _L, _L)], jnp.int32)
                for k in range(_NSEG)
            ]
            t = zero_i
            for b in range(30, 12, -1):
                cand = t | (1 << b)
                cnt = zero_i
                for k in range(_NSEG):
                    cnt = cnt + plsc.all_reduce_population_count(gi[k] >= cand)
                t = jnp.where(cnt >= _K, cand, t)
            collect(lax.bitcast_convert_type(t, jnp.float32))

        ci = [
            lax.bitcast_convert_type(cval2_v[pl.ds(c * _L, _L)], jnp.int32)
            for c in range(ncand)
        ]
        t = zero_i
        for b in range(30, -1, -1):
            cand = t | (1 << b)
            cnt = zero_i
            for c in range(ncand):
                cnt = cnt + plsc.all_reduce_population_count(ci[c] >= cand)
            t = jnp.where(cnt >= _K, cand, t)

        @pl.when(r > 0)
        def _():
            pltpu.make_async_copy(obuf_v, out_hbm.at[row - 1], sem_out).wait()
            for c in range(ncand):
                iv = pidx_v[pl.ds(c * _L, _L)]
                plsc.store_scatter(obuf_v, [iv], zero_f)

        for c in range(ncand):
            iv = cidx2_v[pl.ds(c * _L, _L)]
            keep = ci[c] >= t
            plsc.store_scatter(
                obuf_v, [iv], lax.bitcast_convert_type(ci[c], jnp.float32),
                mask=keep,
            )
        pltpu.make_async_copy(obuf_v, out_hbm.at[row], sem_out).start()
        for c in range(ncand):
            pidx_v[pl.ds(c * _L, _L)] = cidx2_v[pl.ds(c * _L, _L)]
        return t

    init_t = jnp.full((_L,), 0x7F7FFFFF, jnp.int32)
    lax.fori_loop(0, rows_per_w, do_row, init_t)
    pltpu.make_async_copy(
        obuf_v, out_hbm.at[row0 + rows_per_w - 1], sem_out
    ).wait()


@jax.jit
def kernel(x):
    n_rows, n_cols = x.shape
    f = pl.kernel(
        _sc_body,
        out_type=jax.ShapeDtypeStruct((n_rows, n_cols), x.dtype),
        mesh=plsc.VectorSubcoreMesh(
            core_axis_name="c", subcore_axis_name="s",
            num_cores=_NC, num_subcores=_NS,
        ),
        compiler_params=pltpu.CompilerParams(needs_layout_passes=False),
        scratch_types=[
            pltpu.VMEM((2 * n_cols,), jnp.float32),
            pltpu.VMEM((n_cols,), jnp.float32),
            pltpu.VMEM((_NSEG * _L,), jnp.float32),
            pltpu.VMEM((_CAP,), jnp.int32),
            pltpu.VMEM((_CCAP,), jnp.int32),
            pltpu.VMEM((_CCAP,), jnp.float32),
            pltpu.VMEM((_CCAP,), jnp.int32),
            pltpu.SemaphoreType.DMA,
            pltpu.SemaphoreType.DMA,
        ],
    )
    return f(x)


# speedup vs baseline: 1.1123x; 1.1123x over previous
import jax
import jax.numpy as jnp
from jax import lax
from jax.experimental import pallas as pl
from jax.experimental.pallas import tpu as pltpu
from jax.experimental.pallas import tpu_sc as plsc

_K = 64
_L = 16
_NC = 2
_NS = 16
_NW = _NC * _NS
_SLOT = 32
_CAP = _SLOT * _L
_CCAP = 256
_NSEG = 8


def _sc_body(x_hbm, out_hbm, row_v, obuf_v, grp_v, cidx_v, cidx2_v, cval2_v,
             pidx_v, sem_in, sem_out):
    n_rows, n_cols = x_hbm.shape
    n_vregs = n_cols // _L
    rows_per_w = n_rows // _NW
    seg_vregs = n_vregs // _NSEG
    ncand = _CCAP // _L

    wid = lax.axis_index("s") * _NC + lax.axis_index("c")
    row0 = wid * rows_per_w
    iota = lax.iota(jnp.int32, _L)
    zero_f = jnp.zeros((_L,), jnp.float32)
    zero_i = jnp.zeros((_L,), jnp.int32)

    pltpu.make_async_copy(
        x_hbm.at[row0], row_v.at[pl.ds(0, n_cols)], sem_in
    ).start()

    @plsc.parallel_loop(0, n_vregs, unroll=8)
    def _(i):
        obuf_v[pl.ds(i * _L, _L)] = zero_f

    def do_row(r, t_prev):
        row = row0 + r
        base = (r % 2) * n_cols
        nbase = ((r + 1) % 2) * n_cols
        base_splat = zero_i + base
        pltpu.make_async_copy(
            x_hbm.at[row], row_v.at[pl.ds(base, n_cols)], sem_in
        ).wait()

        @pl.when(r < rows_per_w - 1)
        def _():
            pltpu.make_async_copy(
                x_hbm.at[row + 1], row_v.at[pl.ds(nbase, n_cols)], sem_in
            ).start()

        def collect(mthr):
            for c in range(_CAP // _L):
                cidx_v[pl.ds(c * _L, _L)] = zero_i

            @plsc.parallel_loop(0, n_vregs, unroll=8, carry=iota)
            def posv(j, p):
                v = row_v[pl.ds(base + j * _L, _L)]
                m = v >= mthr
                plsc.store_scatter(cidx_v, [p & (_CAP - 1)], iota + j * _L,
                                   mask=m)
                return p + jnp.where(m, _L, 0)

            cl = lax.shift_right_logical(posv - iota, 4)

            for c in range(ncand):
                cidx2_v[pl.ds(c * _L, _L)] = zero_i
                cval2_v[pl.ds(c * _L, _L)] = zero_f

            @plsc.parallel_loop(0, _SLOT, unroll=4, carry=zero_i)
            def cnt2(s, cnt):
                iv = cidx_v[pl.ds(s * _L, _L)]
                valid = cl > s
                vals = plsc.load_gather(row_v, [iv + base_splat])
                pos = cnt + plsc.cumsum(valid.astype(jnp.int32)) - 1
                pos = jnp.minimum(jnp.maximum(pos, 0), _CCAP - 1)
                plsc.store_scatter(cidx2_v, [pos], iv, mask=valid)
                plsc.store_scatter(cval2_v, [pos], vals, mask=valid)
                return cnt + plsc.all_reduce_population_count(valid)

            return cl, cnt2

        m_est = lax.bitcast_convert_type(t_prev, jnp.float32) * 0.9375
        cl, cnt2 = collect(m_est)

        max_cl = jnp.max(cl)
        n_cand = jnp.max(cnt2)
        fb = (max_cl >= _SLOT) | (n_cand < _K) | (n_cand > _CCAP)

        @pl.when(fb)
        def _():
            for seg in range(_NSEG):
                @plsc.parallel_loop(0, seg_vregs, unroll=8, carry=zero_f)
                def acc(i, a):
                    v = row_v[pl.ds(base + (seg * seg_vregs + i) * _L, _L)]
                    return jnp.maximum(a, v)
                grp_v[pl.ds(seg * _L, _L)] = acc

            gi = [
                lax.bitcast_convert_type(grp_v[pl.ds(k * _L, _L)], jnp.int32)
                for k in range(_NSEG)
            ]
            t = zero_i
            for b in range(30, 12, -1):
                cand = t | (1 << b)
                cnt = zero_i
                for k in range(_NSEG):
                    cnt = cnt + plsc.all_reduce_population_count(gi[k] >= cand)
                t = jnp.where(cnt >= _K, cand, t)
            collect(lax.bitcast_convert_type(t, jnp.float32))

        ci = [
            lax.bitcast_convert_type(cval2_v[pl.ds(c * _L, _L)], jnp.int32)
            for c in range(ncand)
        ]
        t = zero_i
        for b in range(30, -1, -1):
            cand = t | (1 << b)
            cnt = zero_i
            for c in range(ncand):
                cnt = cnt + plsc.all_reduce_population_count(ci[c] >= cand)
            t = jnp.where(cnt >= _K, cand, t)

        @pl.when(r > 0)
        def _():
            pltpu.make_async_copy(obuf_v, out_hbm.at[row - 1], sem_out).wait()
            for c in range(ncand):
                iv = pidx_v[pl.ds(c * _L, _L)]
                plsc.store_scatter(obuf_v, [iv], zero_f)

        for c in range(ncand):
            iv = cidx2_v[pl.ds(c * _L, _L)]
            keep = ci[c] >= t
            plsc.store_scatter(
                obuf_v, [iv], lax.bitcast_convert_type(ci[c], jnp.float32),
                mask=keep,
            )
        pltpu.make_async_copy(obuf_v, out_hbm.at[row], sem_out).start()
        for c in range(ncand):
            pidx_v[pl.ds(c * _L, _L)] = cidx2_v[pl.ds(c * _L, _L)]
        return t

    init_t = jnp.full((_L,), 0x7F7FFFFF, jnp.int32)
    lax.fori_loop(0, rows_per_w, do_row, init_t)
    pltpu.make_async_copy(
        obuf_v, out_hbm.at[row0 + rows_per_w - 1], sem_out
    ).wait()


@jax.jit
def kernel(x):
    n_rows, n_cols = x.shape
    f = pl.kernel(
        _sc_body,
        out_type=jax.ShapeDtypeStruct((n_rows, n_cols), x.dtype),
        mesh=plsc.VectorSubcoreMesh(
            core_axis_name="c", subcore_axis_name="s",
            num_cores=_NC, num_subcores=_NS,
        ),
        compiler_params=pltpu.CompilerParams(needs_layout_passes=False),
        scratch_types=[
            pltpu.VMEM((2 * n_cols,), jnp.float32),
            pltpu.VMEM((n_cols,), jnp.float32),
            pltpu.VMEM((_NSEG * _L,), jnp.float32),
            pltpu.VMEM((_CAP,), jnp.int32),
            pltpu.VMEM((_CCAP,), jnp.int32),
            pltpu.VMEM((_CCAP,), jnp.float32),
            pltpu.VMEM((_CCAP,), jnp.int32),
            pltpu.SemaphoreType.DMA,
            pltpu.SemaphoreType.DMA,
        ],
    )
    return f(x)


# speedup vs baseline: 1.1181x; 1.0052x over previous
import jax
import jax.numpy as jnp
from jax import lax
from jax.experimental import pallas as pl
from jax.experimental.pallas import tpu as pltpu
from jax.experimental.pallas import tpu_sc as plsc

_K = 64
_L = 16
_NC = 2
_NS = 16
_NW = _NC * _NS
_SLOT = 32
_CAP = _SLOT * _L
_CCAP = 256
_NSEG = 8


def _sc_body(x_hbm, out_hbm, row_v, obuf_v, grp_v, cidx_v, cidx2_v, cval2_v,
             pidx_v, sem_in, sem_out):
    n_rows, n_cols = x_hbm.shape
    n_vregs = n_cols // _L
    rows_per_w = n_rows // _NW
    seg_vregs = n_vregs // _NSEG
    ncand = _CCAP // _L

    wid = lax.axis_index("s") * _NC + lax.axis_index("c")
    row0 = wid * rows_per_w
    iota = lax.iota(jnp.int32, _L)
    zero_f = jnp.zeros((_L,), jnp.float32)
    zero_i = jnp.zeros((_L,), jnp.int32)

    pltpu.make_async_copy(
        x_hbm.at[row0], row_v.at[pl.ds(0, n_cols)], sem_in
    ).start()

    @plsc.parallel_loop(0, n_vregs, unroll=8)
    def _(i):
        obuf_v[pl.ds(i * _L, _L)] = zero_f

    def do_row(r, t_prev):
        row = row0 + r
        base = (r % 2) * n_cols
        nbase = ((r + 1) % 2) * n_cols
        base_splat = zero_i + base
        pltpu.make_async_copy(
            x_hbm.at[row], row_v.at[pl.ds(base, n_cols)], sem_in
        ).wait()

        @pl.when(r < rows_per_w - 1)
        def _():
            pltpu.make_async_copy(
                x_hbm.at[row + 1], row_v.at[pl.ds(nbase, n_cols)], sem_in
            ).start()

        def collect(mthr):
            for c in range(_CAP // _L):
                cidx_v[pl.ds(c * _L, _L)] = zero_i

            @plsc.parallel_loop(0, n_vregs, unroll=16, carry=iota)
            def posv(j, p):
                v = row_v[pl.ds(base + j * _L, _L)]
                m = v >= mthr
                plsc.store_scatter(cidx_v, [p & (_CAP - 1)], iota + j * _L,
                                   mask=m)
                return p + jnp.where(m, _L, 0)

            cl = lax.shift_right_logical(posv - iota, 4)

            for c in range(ncand):
                cidx2_v[pl.ds(c * _L, _L)] = zero_i
                cval2_v[pl.ds(c * _L, _L)] = zero_f

            @plsc.parallel_loop(0, _SLOT, unroll=8, carry=zero_i)
            def cnt2(s, cnt):
                iv = cidx_v[pl.ds(s * _L, _L)]
                valid = cl > s
                vals = plsc.load_gather(row_v, [iv + base_splat])
                pos = cnt + plsc.cumsum(valid.astype(jnp.int32)) - 1
                pos = jnp.minimum(jnp.maximum(pos, 0), _CCAP - 1)
                plsc.store_scatter(cidx2_v, [pos], iv, mask=valid)
                plsc.store_scatter(cval2_v, [pos], vals, mask=valid)
                return cnt + plsc.all_reduce_population_count(valid)

            return cl, cnt2

        m_est = lax.bitcast_convert_type(t_prev, jnp.float32) * 0.9375
        cl, cnt2 = collect(m_est)

        max_cl = jnp.max(cl)
        n_cand = jnp.max(cnt2)
        fb = (max_cl >= _SLOT) | (n_cand < _K) | (n_cand > _CCAP)

        @pl.when(fb)
        def _():
            for seg in range(_NSEG):
                @plsc.parallel_loop(0, seg_vregs, unroll=8, carry=zero_f)
                def acc(i, a):
                    v = row_v[pl.ds(base + (seg * seg_vregs + i) * _L, _L)]
                    return jnp.maximum(a, v)
                grp_v[pl.ds(seg * _L, _L)] = acc

            gi = [
                lax.bitcast_convert_type(grp_v[pl.ds(k * _L, _L)], jnp.int32)
                for k in range(_NSEG)
            ]
            t = zero_i
            for b in range(30, 12, -1):
                cand = t | (1 << b)
                cnt = zero_i
                for k in range(_NSEG):
                    cnt = cnt + plsc.all_reduce_population_count(gi[k] >= cand)
                t = jnp.where(cnt >= _K, cand, t)
            collect(lax.bitcast_convert_type(t, jnp.float32))

        ci = [
            lax.bitcast_convert_type(cval2_v[pl.ds(c * _L, _L)], jnp.int32)
            for c in range(ncand)
        ]
        t = zero_i
        for b in range(30, -1, -1):
            cand = t | (1 << b)
            cnt = zero_i
            for c in range(ncand):
                cnt = cnt + plsc.all_reduce_population_count(ci[c] >= cand)
            t = jnp.where(cnt >= _K, cand, t)

        @pl.when(r > 0)
        def _():
            pltpu.make_async_copy(obuf_v, out_hbm.at[row - 1], sem_out).wait()
            for c in range(ncand):
                iv = pidx_v[pl.ds(c * _L, _L)]
                plsc.store_scatter(obuf_v, [iv], zero_f)

        for c in range(ncand):
            iv = cidx2_v[pl.ds(c * _L, _L)]
            keep = ci[c] >= t
            plsc.store_scatter(
                obuf_v, [iv], lax.bitcast_convert_type(ci[c], jnp.float32),
                mask=keep,
            )
        pltpu.make_async_copy(obuf_v, out_hbm.at[row], sem_out).start()
        for c in range(ncand):
            pidx_v[pl.ds(c * _L, _L)] = cidx2_v[pl.ds(c * _L, _L)]
        return t

    init_t = jnp.full((_L,), 0x7F7FFFFF, jnp.int32)
    lax.fori_loop(0, rows_per_w, do_row, init_t)
    pltpu.make_async_copy(
        obuf_v, out_hbm.at[row0 + rows_per_w - 1], sem_out
    ).wait()


@jax.jit
def kernel(x):
    n_rows, n_cols = x.shape
    f = pl.kernel(
        _sc_body,
        out_type=jax.ShapeDtypeStruct((n_rows, n_cols), x.dtype),
        mesh=plsc.VectorSubcoreMesh(
            core_axis_name="c", subcore_axis_name="s",
            num_cores=_NC, num_subcores=_NS,
        ),
        compiler_params=pltpu.CompilerParams(needs_layout_passes=False),
        scratch_types=[
            pltpu.VMEM((2 * n_cols,), jnp.float32),
            pltpu.VMEM((n_cols,), jnp.float32),
            pltpu.VMEM((_NSEG * _L,), jnp.float32),
            pltpu.VMEM((_CAP,), jnp.int32),
            pltpu.VMEM((_CCAP,), jnp.int32),
            pltpu.VMEM((_CCAP,), jnp.float32),
            pltpu.VMEM((_CCAP,), jnp.int32),
            pltpu.SemaphoreType.DMA,
            pltpu.SemaphoreType.DMA,
        ],
    )
    return f(x)
